# baseline (speedup 1.0000x reference)
import functools

import numpy as np
import jax
import jax.numpy as jnp
from jax import lax
from jax.experimental import pallas as pl
from jax.experimental.pallas import tpu as pltpu
from jax.experimental.pallas import tpu_sc as plsc

NC = 2
NS = 16
L = 16
C = 128
H = C // 2

CHUNKS_C0 = 80
CHUNKS_C1 = 80

HIMASK = -65536


def _col_perm(d):
    perm = np.empty(d, np.int32)
    for q in range(d // 32):
        for t in range(16):
            perm[32 * q + 2 * t] = 32 * q + t
            perm[32 * q + 2 * t + 1] = 32 * q + 16 + t
    return perm


def _mm_body(x_ref, w_ref, o_ref):
    o_ref[...] = jnp.dot(x_ref[...], w_ref[...],
                         preferred_element_type=jnp.float32
                         ).astype(jnp.bfloat16)


def _finish_body(p_ref, o_ref):
    n = o_ref.shape[0]
    s = p_ref[0, :n, :] + p_ref[1, :n, :]
    o_ref[...] = jnp.where(s >= 0.0, s, 0.01 * s)


def _pack(flat, cap0, a, b, maxc, shape2):
    s0 = flat[:cap0].reshape((NS, a) + shape2)
    s0 = jnp.pad(s0, ((0, 0), (0, maxc - a)) + ((0, 0),) * len(shape2))
    s1 = flat[cap0:].reshape((NS, b) + shape2)
    s1 = jnp.pad(s1, ((0, 0), (0, maxc - b)) + ((0, 0),) * len(shape2))
    return jnp.concatenate([s0, s1], 0)


def kernel(input, edge_index, adj_values, W):
    n, d_in = input.shape
    d_out = W.shape[1]
    e = edge_index.shape[1]
    dq = d_out // 32

    Wp = W[:, _col_perm(d_out)]
    support = pl.pallas_call(
        _mm_body,
        out_shape=jax.ShapeDtypeStruct((n, d_out), jnp.bfloat16),
    )(input, Wp)
    sup_i32 = lax.bitcast_convert_type(
        support.reshape(n, d_out // 2, 2), jnp.int32)

    a, b = CHUNKS_C0, CHUNKS_C1
    maxc = max(a, b)
    cap0 = NS * a * C
    e_pad = NS * (a + b) * C
    pad = e_pad - e
    src = jnp.concatenate([edge_index[1], jnp.zeros((pad,), jnp.int32)])
    dst = jnp.concatenate([edge_index[0], jnp.zeros((pad,), jnp.int32)])
    val = jnp.concatenate([adj_values, jnp.zeros((pad,), jnp.float32)])
    sv3 = jnp.stack(
        [_pack(src, cap0, a, b, maxc, (C,)),
         _pack(lax.bitcast_convert_type(val, jnp.int32),
               cap0, a, b, maxc, (C,))], axis=2)
    dst3 = _pack(dst, cap0, a, b, maxc, (2, H))

    rows_per_tile = pl.cdiv(pl.cdiv(n, NS), C) * C
    n_acc = rows_per_tile * NS
    wb_chunks = rows_per_tile // H

    mesh = plsc.VectorSubcoreMesh(core_axis_name="c", subcore_axis_name="s")

    @functools.partial(
        pl.kernel,
        out_type=jax.ShapeDtypeStruct((NC, n_acc, d_out), jnp.float32),
        mesh=mesh,
        compiler_params=pltpu.CompilerParams(use_tc_tiling_on_sc=False),
        scratch_types=[
            pltpu.VMEM((2, C), jnp.int32),
            pltpu.VMEM((2, C), jnp.int32),
            pltpu.VMEM((2, C), jnp.int32),
            pltpu.VMEM((2, C), jnp.int32),
            pltpu.VMEM((2, H), jnp.int32),
            pltpu.VMEM((2, H), jnp.int32),
            pltpu.VMEM((2, H), jnp.int32),
            pltpu.VMEM((2, H), jnp.int32),
            pltpu.VMEM((C, d_out // 2), jnp.int32),
            pltpu.VMEM((C, d_out // 2), jnp.int32),
            pltpu.VMEM((H, d_out), jnp.float32),
            pltpu.VMEM((H, d_out), jnp.float32),
            pltpu.VMEM_SHARED((n_acc, d_out), jnp.float32),
            pltpu.SemaphoreType.DMA,
            pltpu.SemaphoreType.DMA,
            pltpu.SemaphoreType.DMA,
            pltpu.SemaphoreType.DMA,
            pltpu.SemaphoreType.DMA,
            pltpu.SemaphoreType.DMA,
            pltpu.SemaphoreType.DMA,
            pltpu.SemaphoreType.DMA,
        ],
    )
    def sc_scatter(sup_hbm, src_hbm, dst_hbm, out_hbm,
                   sb0, sb1, sb2, sb3, db0, db1, db2, db3,
                   rw0, rw1, stA, stB, acc_sh,
                   es0, es1, es2, es3, gs0, gs1, ssA, ssB):
        sbufs = [sb0, sb1, sb2, sb3]
        dbufs = [db0, db1, db2, db3]
        rows = [rw0, rw1]
        stages = [stA, stB]
        esem = [es0, es1, es2, es3]
        gsem = [gs0, gs1]
        ssem = [ssA, ssB]

        cid = lax.axis_index("c")
        sid = lax.axis_index("s")
        wid = cid * NS + sid
        row0 = sid * rows_per_tile
        ncc = jnp.where(cid == 0, a, b)
        last = ncc - 1

        def edge_dma(chunk, slot):
            pltpu.async_copy(src_hbm.at[wid, chunk], sbufs[slot], esem[slot])
            pltpu.async_copy(dst_hbm.at[wid, chunk], dbufs[slot], esem[slot])

        def edge_wait(slot):
            pltpu.make_async_copy(src_hbm.at[wid, 0], sbufs[slot],
                                  esem[slot]).wait()
            pltpu.make_async_copy(dst_hbm.at[wid, 0], dbufs[slot],
                                  esem[slot]).wait()

        def gather_dma(slot, rslot):
            pltpu.async_copy(sup_hbm.at[sbufs[slot].at[0, pl.ds(0, H)]],
                             rows[rslot].at[pl.ds(0, H)], gsem[rslot])
            pltpu.async_copy(sup_hbm.at[sbufs[slot].at[0, pl.ds(H, H)]],
                             rows[rslot].at[pl.ds(H, H)], gsem[rslot])

        def gather_wait(rslot):
            pltpu.make_async_copy(sup_hbm.at[sbufs[0].at[0]], rows[rslot],
                                  gsem[rslot]).wait()

        def scatter_dma(slot, h):
            pltpu.async_copy(stages[h], acc_sh.at[dbufs[slot].at[h]],
                             ssem[h], add=True)

        def scatter_wait(h):
            pltpu.make_async_copy(stages[h], acc_sh.at[dbufs[0].at[h]],
                                  ssem[h]).wait()


        @pl.when(ncc > 0)
        def _prologue():
            edge_dma(0, 0)
            edge_dma(jnp.minimum(1, last), 1)
            edge_wait(0)
            gather_dma(0, 0)

        z = jnp.zeros((L,), jnp.float32)

        def zero_body(i, carry):
            for f in range(d_out // L):
                stA[i, pl.ds(f * L, L)] = z
            return carry

        lax.fori_loop(0, H, zero_body, 0)
        for k in range(wb_chunks):
            pltpu.sync_copy(stA, acc_sh.at[pl.ds(row0 + k * H, H)])
        plsc.subcore_barrier()

        @pl.when(ncc > 0)
        def _pipeline():
            edge_dma(0, 0)
            edge_wait(0)
            edge_dma(jnp.minimum(1, last), 1)
            gather_dma(0, 0)

            def half_scale(p, rs, h):
                def scale_group(g, gc):
                    vals = lax.bitcast_convert_type(
                        sbufs[p][1, pl.ds(h * H + g * L, L)], jnp.float32)
                    for ei in range(L):
                        vb = jnp.full((L,), vals[ei], jnp.float32)
                        row = g * L + ei
                        for q in range(dq):
                            x = rows[rs][h * H + row, pl.ds(q * L, L)]
                            lo = lax.bitcast_convert_type(x << 16,
                                                          jnp.float32)
                            hi = lax.bitcast_convert_type(x & HIMASK,
                                                          jnp.float32)
                            stages[h][row, pl.ds(q * 2 * L, L)] = lo * vb
                            stages[h][row, pl.ds((q * 2 + 1) * L, L)] = (
                                hi * vb)
                    return gc

                lax.fori_loop(0, H // L, scale_group, 0)
                scatter_dma(p, h)

            def pipe_body(j4, carry):
                for p in range(4):
                    j = j4 * 4 + p
                    rs = p % 2
                    edge_dma(jnp.minimum(j + 2, last), (p + 2) % 4)
                    edge_wait((p + 1) % 4)
                    gather_dma((p + 1) % 4, 1 - rs)
                    gather_wait(rs)
                    for h in range(2):
                        @pl.when(j > 0)
                        def _():
                            scatter_wait(h)
                        half_scale(p, rs, h)
                return carry

            lax.fori_loop(0, lax.div(ncc, 4), pipe_body, 0)

            edge_wait(1)
            gather_wait(0)
            scatter_wait(0)
            scatter_wait(1)

        plsc.subcore_barrier()

        for k in range(wb_chunks):
            r = row0 + k * H
            pltpu.sync_copy(acc_sh.at[pl.ds(r, H)], stA)
            pltpu.sync_copy(stA, out_hbm.at[cid, pl.ds(r, H)])

    partials = sc_scatter(sup_i32, sv3, dst3)

    return pl.pallas_call(
        _finish_body,
        out_shape=jax.ShapeDtypeStruct((n, d_out), jnp.float32),
    )(partials)


# speedup vs baseline: 1.0016x; 1.0016x over previous
import functools

import numpy as np
import jax
import jax.numpy as jnp
from jax import lax
from jax.experimental import pallas as pl
from jax.experimental.pallas import tpu as pltpu
from jax.experimental.pallas import tpu_sc as plsc

NC = 2
NS = 16
L = 16
C = 128
H = C // 2

CHUNKS_C0 = 80
CHUNKS_C1 = 80

HIMASK = -65536


def _col_perm(d):
    perm = np.empty(d, np.int32)
    for q in range(d // 32):
        for t in range(16):
            perm[32 * q + 2 * t] = 32 * q + t
            perm[32 * q + 2 * t + 1] = 32 * q + 16 + t
    return perm


def _mm_body(x_ref, w_ref, o_ref):
    o_ref[...] = jnp.dot(x_ref[...], w_ref[...],
                         preferred_element_type=jnp.float32
                         ).astype(jnp.bfloat16)


def _finish_body(p_ref, o_ref):
    n = o_ref.shape[0]
    s = p_ref[0, :n, :] + p_ref[1, :n, :]
    o_ref[...] = jnp.where(s >= 0.0, s, 0.01 * s)


def _pack(flat, cap0, a, b, maxc, shape2):
    s0 = flat[:cap0].reshape((NS, a) + shape2)
    s0 = jnp.pad(s0, ((0, 0), (0, maxc - a)) + ((0, 0),) * len(shape2))
    s1 = flat[cap0:].reshape((NS, b) + shape2)
    s1 = jnp.pad(s1, ((0, 0), (0, maxc - b)) + ((0, 0),) * len(shape2))
    return jnp.concatenate([s0, s1], 0)


def kernel(input, edge_index, adj_values, W):
    n, d_in = input.shape
    d_out = W.shape[1]
    e = edge_index.shape[1]
    dq = d_out // 32

    Wp = W[:, _col_perm(d_out)]
    support = pl.pallas_call(
        _mm_body,
        out_shape=jax.ShapeDtypeStruct((n, d_out), jnp.bfloat16),
    )(input, Wp)
    sup_i32 = lax.bitcast_convert_type(
        support.reshape(n, d_out // 2, 2), jnp.int32)

    a, b = CHUNKS_C0, CHUNKS_C1
    maxc = max(a, b)
    cap0 = NS * a * C
    e_pad = NS * (a + b) * C
    pad = e_pad - e
    src = jnp.concatenate([edge_index[1], jnp.zeros((pad,), jnp.int32)])
    dst = jnp.concatenate([edge_index[0], jnp.zeros((pad,), jnp.int32)])
    val = jnp.concatenate([adj_values, jnp.zeros((pad,), jnp.float32)])
    sv3 = jnp.stack(
        [_pack(src, cap0, a, b, maxc, (C,)),
         _pack(lax.bitcast_convert_type(val, jnp.int32),
               cap0, a, b, maxc, (C,))], axis=2)
    dst3 = _pack(dst, cap0, a, b, maxc, (2, H))

    rows_per_tile = pl.cdiv(pl.cdiv(n, NS), C) * C
    n_acc = rows_per_tile * NS
    wb_chunks = rows_per_tile // H

    mesh = plsc.VectorSubcoreMesh(core_axis_name="c", subcore_axis_name="s")

    @functools.partial(
        pl.kernel,
        out_type=jax.ShapeDtypeStruct((NC, n_acc, d_out), jnp.float32),
        mesh=mesh,
        compiler_params=pltpu.CompilerParams(use_tc_tiling_on_sc=False),
        scratch_types=[
            pltpu.VMEM((2, C), jnp.int32),
            pltpu.VMEM((2, C), jnp.int32),
            pltpu.VMEM((2, C), jnp.int32),
            pltpu.VMEM((2, C), jnp.int32),
            pltpu.VMEM((2, H), jnp.int32),
            pltpu.VMEM((2, H), jnp.int32),
            pltpu.VMEM((2, H), jnp.int32),
            pltpu.VMEM((2, H), jnp.int32),
            pltpu.VMEM((C, d_out // 2), jnp.int32),
            pltpu.VMEM((C, d_out // 2), jnp.int32),
            pltpu.VMEM((H, d_out), jnp.float32),
            pltpu.VMEM((H, d_out), jnp.float32),
            pltpu.VMEM_SHARED((n_acc, d_out), jnp.float32),
            pltpu.SemaphoreType.DMA,
            pltpu.SemaphoreType.DMA,
            pltpu.SemaphoreType.DMA,
            pltpu.SemaphoreType.DMA,
            pltpu.SemaphoreType.DMA,
            pltpu.SemaphoreType.DMA,
            pltpu.SemaphoreType.DMA,
            pltpu.SemaphoreType.DMA,
        ],
    )
    def sc_scatter(sup_hbm, src_hbm, dst_hbm, out_hbm,
                   sb0, sb1, sb2, sb3, db0, db1, db2, db3,
                   rw0, rw1, stA, stB, acc_sh,
                   es0, es1, es2, es3, gs0, gs1, ssA, ssB):
        sbufs = [sb0, sb1, sb2, sb3]
        dbufs = [db0, db1, db2, db3]
        rows = [rw0, rw1]
        stages = [stA, stB]
        esem = [es0, es1, es2, es3]
        gsem = [gs0, gs1]
        ssem = [ssA, ssB]

        cid = lax.axis_index("c")
        sid = lax.axis_index("s")
        wid = cid * NS + sid
        row0 = sid * rows_per_tile
        ncc = jnp.where(cid == 0, a, b)
        last = ncc - 1

        def edge_dma(chunk, slot):
            pltpu.async_copy(src_hbm.at[wid, chunk], sbufs[slot], esem[slot])
            pltpu.async_copy(dst_hbm.at[wid, chunk], dbufs[slot], esem[slot])

        def edge_wait(slot):
            pltpu.make_async_copy(src_hbm.at[wid, 0], sbufs[slot],
                                  esem[slot]).wait()
            pltpu.make_async_copy(dst_hbm.at[wid, 0], dbufs[slot],
                                  esem[slot]).wait()

        def gather_dma(slot, rslot):
            pltpu.async_copy(sup_hbm.at[sbufs[slot].at[0]], rows[rslot],
                             gsem[rslot])

        def gather_wait(rslot):
            pltpu.make_async_copy(sup_hbm.at[sbufs[0].at[0]], rows[rslot],
                                  gsem[rslot]).wait()

        def scatter_dma(slot, h):
            pltpu.async_copy(stages[h], acc_sh.at[dbufs[slot].at[h]],
                             ssem[h], add=True)

        def scatter_wait(h):
            pltpu.make_async_copy(stages[h], acc_sh.at[dbufs[0].at[h]],
                                  ssem[h]).wait()


        @pl.when(ncc > 0)
        def _prologue():
            edge_dma(0, 0)
            edge_dma(jnp.minimum(1, last), 1)
            edge_wait(0)
            gather_dma(0, 0)

        z = jnp.zeros((L,), jnp.float32)

        def zero_body(i, carry):
            for f in range(d_out // L):
                stA[i, pl.ds(f * L, L)] = z
            return carry

        lax.fori_loop(0, H, zero_body, 0)
        for k in range(wb_chunks):
            pltpu.sync_copy(stA, acc_sh.at[pl.ds(row0 + k * H, H)])
        plsc.subcore_barrier()

        @pl.when(ncc > 0)
        def _pipeline():
            edge_dma(0, 0)
            edge_wait(0)
            edge_dma(jnp.minimum(1, last), 1)
            gather_dma(0, 0)

            def half_scale(p, rs, h):
                def scale_group(g, gc):
                    vals = lax.bitcast_convert_type(
                        sbufs[p][1, pl.ds(h * H + g * L, L)], jnp.float32)
                    for ei in range(L):
                        vb = jnp.full((L,), vals[ei], jnp.float32)
                        row = g * L + ei
                        for q in range(dq):
                            x = rows[rs][h * H + row, pl.ds(q * L, L)]
                            lo = lax.bitcast_convert_type(x << 16,
                                                          jnp.float32)
                            hi = lax.bitcast_convert_type(x & HIMASK,
                                                          jnp.float32)
                            stages[h][row, pl.ds(q * 2 * L, L)] = lo * vb
                            stages[h][row, pl.ds((q * 2 + 1) * L, L)] = (
                                hi * vb)
                    return gc

                lax.fori_loop(0, H // L, scale_group, 0)
                scatter_dma(p, h)

            def pipe_body(j4, carry):
                for p in range(4):
                    j = j4 * 4 + p
                    rs = p % 2
                    edge_dma(jnp.minimum(j + 2, last), (p + 2) % 4)
                    edge_wait((p + 1) % 4)
                    gather_dma((p + 1) % 4, 1 - rs)
                    gather_wait(rs)
                    for h in range(2):
                        @pl.when(j > 0)
                        def _():
                            scatter_wait(h)
                        half_scale(p, rs, h)
                return carry

            lax.fori_loop(0, lax.div(ncc, 4), pipe_body, 0)

            edge_wait(1)
            gather_wait(0)
            scatter_wait(0)
            scatter_wait(1)

        plsc.subcore_barrier()

        for k in range(wb_chunks):
            r = row0 + k * H
            pltpu.sync_copy(acc_sh.at[pl.ds(r, H)], stA)
            pltpu.sync_copy(stA, out_hbm.at[cid, pl.ds(r, H)])

    partials = sc_scatter(sup_i32, sv3, dst3)

    return pl.pallas_call(
        _finish_body,
        out_shape=jax.ShapeDtypeStruct((n, d_out), jnp.float32),
    )(partials)


# speedup vs baseline: 1.0423x; 1.0406x over previous
import functools

import numpy as np
import jax
import jax.numpy as jnp
from jax import lax
from jax.experimental import pallas as pl
from jax.experimental.pallas import tpu as pltpu
from jax.experimental.pallas import tpu_sc as plsc

NC = 2
NS = 16
L = 16
C = 128
H = C // 2

CHUNKS_C0 = 80
CHUNKS_C1 = 80

HIMASK = -65536


def _col_perm(d):
    perm = np.empty(d, np.int32)
    for q in range(d // 32):
        for t in range(16):
            perm[32 * q + 2 * t] = 32 * q + t
            perm[32 * q + 2 * t + 1] = 32 * q + 16 + t
    return perm


def _mm_body(x_ref, w_ref, o_ref):
    o_ref[...] = jnp.dot(x_ref[...], w_ref[...],
                         preferred_element_type=jnp.float32
                         ).astype(jnp.bfloat16)


def _finish_body(p_ref, o_ref):
    n = o_ref.shape[0]
    s = p_ref[0, :n, :] + p_ref[1, :n, :]
    o_ref[...] = jnp.where(s >= 0.0, s, 0.01 * s)


def _pack(flat, cap0, a, b, maxc, shape2):
    s0 = flat[:cap0].reshape((NS, a) + shape2)
    s0 = jnp.pad(s0, ((0, 0), (0, maxc - a)) + ((0, 0),) * len(shape2))
    s1 = flat[cap0:].reshape((NS, b) + shape2)
    s1 = jnp.pad(s1, ((0, 0), (0, maxc - b)) + ((0, 0),) * len(shape2))
    return jnp.concatenate([s0, s1], 0)


def kernel(input, edge_index, adj_values, W):
    n, d_in = input.shape
    d_out = W.shape[1]
    e = edge_index.shape[1]
    dq = d_out // 32

    Wp = W[:, _col_perm(d_out)]
    support = pl.pallas_call(
        _mm_body,
        out_shape=jax.ShapeDtypeStruct((n, d_out), jnp.bfloat16),
    )(input, Wp)
    sup_i32 = lax.bitcast_convert_type(
        support.reshape(n, d_out // 2, 2), jnp.int32)

    a, b = CHUNKS_C0, CHUNKS_C1
    maxc = max(a, b)
    cap0 = NS * a * C
    e_pad = NS * (a + b) * C
    pad = e_pad - e
    src = jnp.concatenate([edge_index[1], jnp.zeros((pad,), jnp.int32)])
    dst = jnp.concatenate([edge_index[0], jnp.zeros((pad,), jnp.int32)])
    val = jnp.concatenate([adj_values, jnp.zeros((pad,), jnp.float32)])
    sv = jnp.stack([src, lax.bitcast_convert_type(val, jnp.int32)], 1)
    sv3 = _pack(sv.reshape(-1), cap0 * 2, a, b, maxc, (C, 2))
    sv3 = jnp.swapaxes(sv3.reshape(NC * NS, maxc, C, 2), 2, 3)
    dst3 = _pack(dst, cap0, a, b, maxc, (2, H))

    rows_per_tile = pl.cdiv(pl.cdiv(n, NS), C) * C
    n_acc = rows_per_tile * NS
    wb_chunks = rows_per_tile // H

    mesh = plsc.VectorSubcoreMesh(core_axis_name="c", subcore_axis_name="s")

    @functools.partial(
        pl.kernel,
        out_type=jax.ShapeDtypeStruct((NC, n_acc, d_out), jnp.float32),
        mesh=mesh,
        compiler_params=pltpu.CompilerParams(use_tc_tiling_on_sc=False),
        scratch_types=[
            pltpu.VMEM((2, C), jnp.int32),
            pltpu.VMEM((2, C), jnp.int32),
            pltpu.VMEM((2, C), jnp.int32),
            pltpu.VMEM((2, C), jnp.int32),
            pltpu.VMEM((2, H), jnp.int32),
            pltpu.VMEM((2, H), jnp.int32),
            pltpu.VMEM((2, H), jnp.int32),
            pltpu.VMEM((2, H), jnp.int32),
            pltpu.VMEM((C, d_out // 2), jnp.int32),
            pltpu.VMEM((C, d_out // 2), jnp.int32),
            pltpu.VMEM((H, d_out), jnp.float32),
            pltpu.VMEM((H, d_out), jnp.float32),
            pltpu.VMEM_SHARED((n_acc, d_out), jnp.float32),
            pltpu.SemaphoreType.DMA,
            pltpu.SemaphoreType.DMA,
            pltpu.SemaphoreType.DMA,
            pltpu.SemaphoreType.DMA,
            pltpu.SemaphoreType.DMA,
            pltpu.SemaphoreType.DMA,
            pltpu.SemaphoreType.DMA,
            pltpu.SemaphoreType.DMA,
        ],
    )
    def sc_scatter(sup_hbm, src_hbm, dst_hbm, out_hbm,
                   sb0, sb1, sb2, sb3, db0, db1, db2, db3,
                   rw0, rw1, stA, stB, acc_sh,
                   es0, es1, es2, es3, gs0, gs1, ssA, ssB):
        sbufs = [sb0, sb1, sb2, sb3]
        dbufs = [db0, db1, db2, db3]
        rows = [rw0, rw1]
        stages = [stA, stB]
        esem = [es0, es1, es2, es3]
        gsem = [gs0, gs1]
        ssem = [ssA, ssB]

        cid = lax.axis_index("c")
        sid = lax.axis_index("s")
        wid = cid * NS + sid
        row0 = sid * rows_per_tile
        ncc = jnp.where(cid == 0, a, b)
        last = ncc - 1

        def edge_dma(chunk, slot):
            pltpu.async_copy(src_hbm.at[wid, chunk], sbufs[slot], esem[slot])
            pltpu.async_copy(dst_hbm.at[wid, chunk], dbufs[slot], esem[slot])

        def edge_wait(slot):
            pltpu.make_async_copy(src_hbm.at[wid, 0], sbufs[slot],
                                  esem[slot]).wait()
            pltpu.make_async_copy(dst_hbm.at[wid, 0], dbufs[slot],
                                  esem[slot]).wait()

        def gather_dma(slot, rslot):
            pltpu.async_copy(sup_hbm.at[sbufs[slot].at[0]], rows[rslot],
                             gsem[rslot])

        def gather_wait(rslot):
            pltpu.make_async_copy(sup_hbm.at[sbufs[0].at[0]], rows[rslot],
                                  gsem[rslot]).wait()

        def scatter_dma(slot, h):
            pltpu.async_copy(stages[h], acc_sh.at[dbufs[slot].at[h]],
                             ssem[h], add=True)

        def scatter_wait(h):
            pltpu.make_async_copy(stages[h], acc_sh.at[dbufs[0].at[h]],
                                  ssem[h]).wait()


        @pl.when(ncc > 0)
        def _prologue():
            edge_dma(0, 0)
            edge_dma(jnp.minimum(1, last), 1)
            edge_wait(0)
            gather_dma(0, 0)

        z = jnp.zeros((L,), jnp.float32)

        def zero_body(i, carry):
            for f in range(d_out // L):
                stA[i, pl.ds(f * L, L)] = z
            return carry

        lax.fori_loop(0, H, zero_body, 0)
        for k in range(wb_chunks):
            pltpu.sync_copy(stA, acc_sh.at[pl.ds(row0 + k * H, H)])
        plsc.subcore_barrier()

        @pl.when(ncc > 0)
        def _pipeline():
            edge_dma(0, 0)
            edge_wait(0)
            edge_dma(jnp.minimum(1, last), 1)
            gather_dma(0, 0)

            def half_scale(p, rs, h):
                def scale_group(g, gc):
                    vals = lax.bitcast_convert_type(
                        sbufs[p][1, pl.ds(h * H + g * L, L)], jnp.float32)
                    for ei in range(L):
                        vb = jnp.full((L,), vals[ei], jnp.float32)
                        row = g * L + ei
                        for q in range(dq):
                            x = rows[rs][h * H + row, pl.ds(q * L, L)]
                            lo = lax.bitcast_convert_type(x << 16,
                                                          jnp.float32)
                            hi = lax.bitcast_convert_type(x & HIMASK,
                                                          jnp.float32)
                            stages[h][row, pl.ds(q * 2 * L, L)] = lo * vb
                            stages[h][row, pl.ds((q * 2 + 1) * L, L)] = (
                                hi * vb)
                    return gc

                lax.fori_loop(0, H // L, scale_group, 0)
                scatter_dma(p, h)

            def pipe_body(j4, carry):
                for p in range(4):
                    j = j4 * 4 + p
                    rs = p % 2
                    edge_dma(jnp.minimum(j + 2, last), (p + 2) % 4)
                    edge_wait((p + 1) % 4)
                    gather_dma((p + 1) % 4, 1 - rs)
                    gather_wait(rs)
                    for h in range(2):
                        @pl.when(j > 0)
                        def _():
                            scatter_wait(h)
                        half_scale(p, rs, h)
                return carry

            lax.fori_loop(0, lax.div(ncc, 4), pipe_body, 0)

            edge_wait(1)
            gather_wait(0)
            scatter_wait(0)
            scatter_wait(1)

        plsc.subcore_barrier()

        for k in range(wb_chunks):
            r = row0 + k * H
            pltpu.sync_copy(acc_sh.at[pl.ds(r, H)], stA)
            pltpu.sync_copy(stA, out_hbm.at[cid, pl.ds(r, H)])

    partials = sc_scatter(sup_i32, sv3, dst3)

    return pl.pallas_call(
        _finish_body,
        out_shape=jax.ShapeDtypeStruct((n, d_out), jnp.float32),
    )(partials)


# speedup vs baseline: 1.0492x; 1.0066x over previous
import functools

import numpy as np
import jax
import jax.numpy as jnp
from jax import lax
from jax.experimental import pallas as pl
from jax.experimental.pallas import tpu as pltpu
from jax.experimental.pallas import tpu_sc as plsc

NC = 2
NS = 16
L = 16
C = 128
H = C // 2

CHUNKS_C0 = 80
CHUNKS_C1 = 80

HIMASK = -65536


def _col_perm(d):
    perm = np.empty(d, np.int32)
    for q in range(d // 32):
        for t in range(16):
            perm[32 * q + 2 * t] = 32 * q + t
            perm[32 * q + 2 * t + 1] = 32 * q + 16 + t
    return perm


def _mm_body(x_ref, w_ref, o_ref):
    o_ref[...] = jnp.dot(x_ref[...], w_ref[...],
                         preferred_element_type=jnp.float32
                         ).astype(jnp.bfloat16)


def _finish_body(p_ref, o_ref):
    n = o_ref.shape[0]
    s = p_ref[0, :n, :] + p_ref[1, :n, :]
    o_ref[...] = jnp.where(s >= 0.0, s, 0.01 * s)


def _pack(flat, cap0, a, b, maxc, shape2):
    s0 = flat[:cap0].reshape((NS, a) + shape2)
    s0 = jnp.pad(s0, ((0, 0), (0, maxc - a)) + ((0, 0),) * len(shape2))
    s1 = flat[cap0:].reshape((NS, b) + shape2)
    s1 = jnp.pad(s1, ((0, 0), (0, maxc - b)) + ((0, 0),) * len(shape2))
    return jnp.concatenate([s0, s1], 0)


def kernel(input, edge_index, adj_values, W):
    n, d_in = input.shape
    d_out = W.shape[1]
    e = edge_index.shape[1]
    dq = d_out // 32

    Wp = W[:, _col_perm(d_out)]
    support = pl.pallas_call(
        _mm_body,
        out_shape=jax.ShapeDtypeStruct((n, d_out), jnp.bfloat16),
    )(input, Wp)
    sup_i32 = lax.bitcast_convert_type(
        support.reshape(n, d_out // 2, 2), jnp.int32)

    a, b = CHUNKS_C0, CHUNKS_C1
    maxc = max(a, b)
    cap0 = NS * a * C
    e_pad = NS * (a + b) * C
    pad = e_pad - e
    src = jnp.concatenate([edge_index[1], jnp.zeros((pad,), jnp.int32)])
    dst = jnp.concatenate([edge_index[0], jnp.zeros((pad,), jnp.int32)])
    val = jnp.concatenate([adj_values, jnp.zeros((pad,), jnp.float32)])
    sv = jnp.stack([src, lax.bitcast_convert_type(val, jnp.int32)], 1)
    sv3 = _pack(sv.reshape(-1), cap0 * 2, a, b, maxc, (C, 2))
    sv3 = jnp.swapaxes(sv3.reshape(NC * NS, maxc, C, 2), 2, 3)
    dst3 = _pack(dst, cap0, a, b, maxc, (2, H))

    rows_per_tile = pl.cdiv(pl.cdiv(n, NS), C) * C
    n_acc = rows_per_tile * NS
    wb_chunks = rows_per_tile // H

    mesh = plsc.VectorSubcoreMesh(core_axis_name="c", subcore_axis_name="s")

    @functools.partial(
        pl.kernel,
        out_type=jax.ShapeDtypeStruct((NC, n_acc, d_out), jnp.float32),
        mesh=mesh,
        compiler_params=pltpu.CompilerParams(use_tc_tiling_on_sc=False),
        scratch_types=[
            pltpu.VMEM((2, C), jnp.int32),
            pltpu.VMEM((2, C), jnp.int32),
            pltpu.VMEM((2, C), jnp.int32),
            pltpu.VMEM((2, C), jnp.int32),
            pltpu.VMEM((2, H), jnp.int32),
            pltpu.VMEM((2, H), jnp.int32),
            pltpu.VMEM((2, H), jnp.int32),
            pltpu.VMEM((2, H), jnp.int32),
            pltpu.VMEM((C, d_out // 2), jnp.int32),
            pltpu.VMEM((C, d_out // 2), jnp.int32),
            pltpu.VMEM((H, d_out), jnp.float32),
            pltpu.VMEM((H, d_out), jnp.float32),
            pltpu.VMEM_SHARED((n_acc, d_out), jnp.float32),
            pltpu.SemaphoreType.DMA,
            pltpu.SemaphoreType.DMA,
            pltpu.SemaphoreType.DMA,
            pltpu.SemaphoreType.DMA,
            pltpu.SemaphoreType.DMA,
            pltpu.SemaphoreType.DMA,
            pltpu.SemaphoreType.DMA,
            pltpu.SemaphoreType.DMA,
        ],
    )
    def sc_scatter(sup_hbm, src_hbm, dst_hbm, out_hbm,
                   sb0, sb1, sb2, sb3, db0, db1, db2, db3,
                   rw0, rw1, stA, stB, acc_sh,
                   es0, es1, es2, es3, gs0, gs1, ssA, ssB):
        sbufs = [sb0, sb1, sb2, sb3]
        dbufs = [db0, db1, db2, db3]
        rows = [rw0, rw1]
        stages = [stA, stB]
        esem = [es0, es1, es2, es3]
        gsem = [gs0, gs1]
        ssem = [ssA, ssB]

        cid = lax.axis_index("c")
        sid = lax.axis_index("s")
        wid = cid * NS + sid
        row0 = sid * rows_per_tile
        ncc = jnp.where(cid == 0, a, b)
        last = ncc - 1

        def edge_dma(chunk, slot):
            pltpu.async_copy(src_hbm.at[wid, chunk], sbufs[slot], esem[slot])
            pltpu.async_copy(dst_hbm.at[wid, chunk], dbufs[slot], esem[slot])

        def edge_wait(slot):
            pltpu.make_async_copy(src_hbm.at[wid, 0], sbufs[slot],
                                  esem[slot]).wait()
            pltpu.make_async_copy(dst_hbm.at[wid, 0], dbufs[slot],
                                  esem[slot]).wait()

        def gather_dma(slot, rslot):
            pltpu.async_copy(sup_hbm.at[sbufs[slot].at[0]], rows[rslot],
                             gsem[rslot])

        def gather_wait(rslot):
            pltpu.make_async_copy(sup_hbm.at[sbufs[0].at[0]], rows[rslot],
                                  gsem[rslot]).wait()

        def scatter_dma(slot, h):
            pltpu.async_copy(stages[h], acc_sh.at[dbufs[slot].at[h]],
                             ssem[h], add=True)

        def scatter_wait(h):
            pltpu.make_async_copy(stages[h], acc_sh.at[dbufs[0].at[h]],
                                  ssem[h]).wait()

        @pl.when(ncc > 0)
        def _prologue():
            edge_dma(0, 0)
            edge_dma(jnp.minimum(1, last), 1)
            edge_wait(0)
            gather_dma(0, 0)

        z = jnp.zeros((L,), jnp.float32)

        def zero_body(i, carry):
            for f in range(d_out // L):
                stA[i, pl.ds(f * L, L)] = z
            return carry

        lax.fori_loop(0, H, zero_body, 0)
        for k in range(wb_chunks):
            pltpu.sync_copy(stA, acc_sh.at[pl.ds(row0 + k * H, H)])
        plsc.subcore_barrier()

        @pl.when(ncc > 0)
        def _pipeline():
            def half_scale(p, rs, h):
                def scale_group(g, gc):
                    vals = lax.bitcast_convert_type(
                        sbufs[p][1, pl.ds(h * H + g * L, L)], jnp.float32)
                    for ei in range(L):
                        vb = jnp.full((L,), vals[ei], jnp.float32)
                        row = g * L + ei
                        for q in range(dq):
                            x = rows[rs][h * H + row, pl.ds(q * L, L)]
                            lo = lax.bitcast_convert_type(x << 16,
                                                          jnp.float32)
                            hi = lax.bitcast_convert_type(x & HIMASK,
                                                          jnp.float32)
                            stages[h][row, pl.ds(q * 2 * L, L)] = lo * vb
                            stages[h][row, pl.ds((q * 2 + 1) * L, L)] = (
                                hi * vb)
                    return gc

                lax.fori_loop(0, H // L, scale_group, 0)
                scatter_dma(p, h)

            def pipe_body(j4, carry):
                for p in range(4):
                    j = j4 * 4 + p
                    rs = p % 2
                    edge_dma(jnp.minimum(j + 2, last), (p + 2) % 4)
                    edge_wait((p + 1) % 4)
                    gather_dma((p + 1) % 4, 1 - rs)
                    gather_wait(rs)
                    for h in range(2):
                        @pl.when(j > 0)
                        def _():
                            scatter_wait(h)
                        half_scale(p, rs, h)
                return carry

            lax.fori_loop(0, lax.div(ncc, 4), pipe_body, 0)

            edge_wait(1)
            gather_wait(0)
            scatter_wait(0)
            scatter_wait(1)

        plsc.subcore_barrier()

        for k in range(wb_chunks):
            r = row0 + k * H
            pltpu.sync_copy(acc_sh.at[pl.ds(r, H)], stA)
            pltpu.sync_copy(stA, out_hbm.at[cid, pl.ds(r, H)])

    partials = sc_scatter(sup_i32, sv3, dst3)

    return pl.pallas_call(
        _finish_body,
        out_shape=jax.ShapeDtypeStruct((n, d_out), jnp.float32),
    )(partials)
